# Initial kernel scaffold; baseline (speedup 1.0000x reference)
import jax
import jax.numpy as jnp
from jax.experimental import pallas as pl


def kernel(queries, targets):
    raise NotImplementedError("write your pallas kernel here")


# speedup vs baseline: 45.3723x; 45.3723x over previous
import functools

import jax
import jax.numpy as jnp
from jax.experimental import pallas as pl

DELTA = 1.0
ROW_BLOCK = 256


def _mine_kernel(q_row_ref, q_all_ref, t_row_ref, t_all_ref, out_ref):
    i = pl.program_id(0)
    q_row = q_row_ref[...]
    q_all = q_all_ref[...]
    t_row = t_row_ref[...]
    t_all = t_all_ref[...]

    R = q_row.shape[0]
    B = q_all.shape[0]

    g = jax.lax.dot_general(
        q_row, q_all,
        dimension_numbers=(((1,), (1,)), ((), ())),
        preferred_element_type=jnp.float32,
    )
    n_row = jnp.sum(q_row * q_row, axis=1, keepdims=True)
    n_all = jnp.sum(q_all * q_all, axis=1, keepdims=True).reshape(1, B)
    d = jnp.maximum(n_row + n_all - 2.0 * g, 0.0)

    row_ids = jax.lax.broadcasted_iota(jnp.int32, (R, B), 0) + i * R
    col_ids = jax.lax.broadcasted_iota(jnp.int32, (R, B), 1)
    diag = row_ids == col_ids
    same = t_row == t_all

    inf = jnp.float32(jnp.inf)
    big_idx = jnp.int32(B)

    masked_pos = jnp.where(diag, -inf, jnp.where(same, d, 0.0))
    m_pos = jnp.max(masked_pos, axis=1, keepdims=True)
    jp = jnp.min(jnp.where(masked_pos == m_pos, col_ids, big_idx),
                 axis=1, keepdims=True)
    d_pos = jnp.sum(jnp.where(col_ids == jp, d, 0.0), axis=1)

    masked_neg = jnp.where(diag | same, inf, d)
    m_neg = jnp.min(masked_neg, axis=1, keepdims=True)
    jn = jnp.min(jnp.where(masked_neg == m_neg, col_ids, big_idx),
                 axis=1, keepdims=True)
    d_neg = jnp.sum(jnp.where(col_ids == jn, d, 0.0), axis=1)

    out_ref[...] = jnp.maximum(DELTA - d_pos + d_neg, 0.0)


@jax.jit
def kernel(queries, targets):
    B, F = queries.shape
    t = targets.astype(jnp.int32)
    t_col = t.reshape(B, 1)
    t_row = t.reshape(1, B)
    num_blocks = B // ROW_BLOCK

    per_row = pl.pallas_call(
        _mine_kernel,
        grid=(num_blocks,),
        in_specs=[
            pl.BlockSpec((ROW_BLOCK, F), lambda i: (i, 0)),
            pl.BlockSpec((B, F), lambda i: (0, 0)),
            pl.BlockSpec((ROW_BLOCK, 1), lambda i: (i, 0)),
            pl.BlockSpec((1, B), lambda i: (0, 0)),
        ],
        out_specs=pl.BlockSpec((ROW_BLOCK,), lambda i: (i,)),
        out_shape=jax.ShapeDtypeStruct((B,), jnp.float32),
    )(queries, queries, t_col, t_row)

    return jnp.mean(per_row)


# speedup vs baseline: 84.9516x; 1.8723x over previous
import jax
import jax.numpy as jnp
from jax.experimental import pallas as pl
from jax.experimental.pallas import tpu as pltpu

DELTA = 1.0
ROW_BLOCK = 256


def _mine_kernel(q_row_ref, q_all_ref, t_row_ref, t_all_ref, out_ref,
                 nall_ref):
    i = pl.program_id(0)
    B = q_all_ref.shape[0]
    R = q_row_ref.shape[0]

    @pl.when(i == 0)
    def _init_norms():
        qa = q_all_ref[...]
        nall_ref[...] = jnp.sum(qa * qa, axis=1).reshape(1, B)

    q_row = q_row_ref[...]
    n_row = jnp.sum(q_row * q_row, axis=1, keepdims=True)
    g = jax.lax.dot_general(
        q_row, q_all_ref[...],
        dimension_numbers=(((1,), (1,)), ((), ())),
        preferred_element_type=jnp.float32,
    )
    d = jnp.maximum(n_row + nall_ref[...] - 2.0 * g, 0.0)

    same = t_row_ref[...] == t_all_ref[...]
    row_ids = jax.lax.broadcasted_iota(jnp.int32, (R, B), 0) + i * R
    col_ids = jax.lax.broadcasted_iota(jnp.int32, (R, B), 1)
    diag = row_ids == col_ids

    inf = jnp.float32(jnp.inf)
    m_pos = jnp.max(jnp.where(same & ~diag, d, 0.0), axis=1, keepdims=True)
    m_neg = jnp.min(jnp.where(same, inf, d), axis=1, keepdims=True)

    out_ref[...] = jnp.maximum(DELTA - m_pos + m_neg, 0.0)

    ok = (jnp.min(m_pos) > 0.0) & (jnp.max(m_neg) < inf)

    @pl.when(jnp.logical_not(ok))
    def _exact_fix():
        big_idx = jnp.int32(B)
        masked_pos = jnp.where(diag, -inf, jnp.where(same, d, 0.0))
        mp = jnp.max(masked_pos, axis=1, keepdims=True)
        jp = jnp.min(jnp.where(masked_pos == mp, col_ids, big_idx),
                     axis=1, keepdims=True)
        d_pos = jnp.sum(jnp.where(col_ids == jp, d, 0.0), axis=1,
                        keepdims=True)
        masked_neg = jnp.where(diag | same, inf, d)
        mn = jnp.min(masked_neg, axis=1, keepdims=True)
        jn = jnp.min(jnp.where(masked_neg == mn, col_ids, big_idx),
                     axis=1, keepdims=True)
        d_neg = jnp.sum(jnp.where(col_ids == jn, d, 0.0), axis=1,
                        keepdims=True)
        out_ref[...] = jnp.maximum(DELTA - d_pos + d_neg, 0.0)


@jax.jit
def kernel(queries, targets):
    B, F = queries.shape
    t = targets.astype(jnp.int32)
    num_blocks = B // ROW_BLOCK

    per_row = pl.pallas_call(
        _mine_kernel,
        grid=(num_blocks,),
        in_specs=[
            pl.BlockSpec((ROW_BLOCK, F), lambda i: (i, 0)),
            pl.BlockSpec((B, F), lambda i: (0, 0)),
            pl.BlockSpec((ROW_BLOCK, 1), lambda i: (i, 0)),
            pl.BlockSpec((1, B), lambda i: (0, 0)),
        ],
        out_specs=pl.BlockSpec((ROW_BLOCK, 1), lambda i: (i, 0)),
        out_shape=jax.ShapeDtypeStruct((B, 1), jnp.float32),
        scratch_shapes=[pltpu.VMEM((1, B), jnp.float32)],
    )(queries, queries, t.reshape(B, 1), t.reshape(1, B))

    return jnp.mean(per_row)


# speedup vs baseline: 85.0740x; 1.0014x over previous
import jax
import jax.numpy as jnp
from jax.experimental import pallas as pl
from jax.experimental.pallas import tpu as pltpu

DELTA = 1.0
ROW_BLOCK = 256


def _prep_kernel(q_ref, a_ref, c_ref):
    q = q_ref[...]
    n = jnp.sum(q * q, axis=1, keepdims=True)
    F = q.shape[1]
    a_ref[:, :F] = q
    a_ref[:, F:F + 1] = n
    a_ref[:, F + 1:F + 2] = jnp.ones_like(n)
    c_ref[:, :F] = -2.0 * q
    c_ref[:, F:F + 1] = jnp.ones_like(n)
    c_ref[:, F + 1:F + 2] = n


def _mine_kernel(a_row_ref, c_all_ref, t_row_ref, t_all_ref, out_ref):
    i = pl.program_id(0)
    B = c_all_ref.shape[0]
    R = a_row_ref.shape[0]

    d = jax.lax.dot_general(
        a_row_ref[...], c_all_ref[...],
        dimension_numbers=(((1,), (1,)), ((), ())),
        preferred_element_type=jnp.float32,
    )

    same = t_row_ref[...] == t_all_ref[...]
    rid = jax.lax.broadcasted_iota(jnp.int32, (R, 1), 0) + i * R
    col_ids = jax.lax.broadcasted_iota(jnp.int32, (R, B), 1)
    diag = col_ids == rid

    inf = jnp.float32(jnp.inf)
    m_pos = jnp.max(jnp.where(same & ~diag, d, 0.0), axis=1, keepdims=True)
    m_neg = jnp.min(jnp.where(same, inf, d), axis=1, keepdims=True)

    out_ref[...] = jnp.maximum(DELTA - m_pos + m_neg, 0.0)

    ok = (jnp.min(m_pos) > 0.0) & (jnp.max(m_neg) < inf)

    @pl.when(jnp.logical_not(ok))
    def _exact_fix():
        dc = jnp.maximum(d, 0.0)
        big_idx = jnp.int32(B)
        masked_pos = jnp.where(diag, -inf, jnp.where(same, dc, 0.0))
        mp = jnp.max(masked_pos, axis=1, keepdims=True)
        jp = jnp.min(jnp.where(masked_pos == mp, col_ids, big_idx),
                     axis=1, keepdims=True)
        d_pos = jnp.sum(jnp.where(col_ids == jp, dc, 0.0), axis=1,
                        keepdims=True)
        masked_neg = jnp.where(diag | same, inf, dc)
        mn = jnp.min(masked_neg, axis=1, keepdims=True)
        jn = jnp.min(jnp.where(masked_neg == mn, col_ids, big_idx),
                     axis=1, keepdims=True)
        d_neg = jnp.sum(jnp.where(col_ids == jn, dc, 0.0), axis=1,
                        keepdims=True)
        out_ref[...] = jnp.maximum(DELTA - d_pos + d_neg, 0.0)


@jax.jit
def kernel(queries, targets):
    B, F = queries.shape
    FA = F + 2
    t = targets.astype(jnp.int32)
    num_blocks = B // ROW_BLOCK

    a_aug, c_aug = pl.pallas_call(
        _prep_kernel,
        out_shape=(
            jax.ShapeDtypeStruct((B, FA), jnp.float32),
            jax.ShapeDtypeStruct((B, FA), jnp.float32),
        ),
    )(queries)

    per_row = pl.pallas_call(
        _mine_kernel,
        grid=(num_blocks,),
        in_specs=[
            pl.BlockSpec((ROW_BLOCK, FA), lambda i: (i, 0)),
            pl.BlockSpec((B, FA), lambda i: (0, 0)),
            pl.BlockSpec((ROW_BLOCK, 1), lambda i: (i, 0)),
            pl.BlockSpec((1, B), lambda i: (0, 0)),
        ],
        out_specs=pl.BlockSpec((ROW_BLOCK, 1), lambda i: (i, 0)),
        out_shape=jax.ShapeDtypeStruct((B, 1), jnp.float32),
        compiler_params=pltpu.CompilerParams(
            dimension_semantics=("parallel",),
        ),
    )(a_aug, c_aug, t.reshape(B, 1), t.reshape(1, B))

    return jnp.mean(per_row)


# speedup vs baseline: 100.6601x; 1.1832x over previous
import jax
import jax.numpy as jnp
from jax.experimental import pallas as pl
from jax.experimental.pallas import tpu as pltpu

DELTA = 1.0
ROW_BLOCK = 256


def _prep_kernel(q_ref, a_ref, c_ref):
    q = q_ref[...]
    n = jnp.sum(q * q, axis=1, keepdims=True)
    F = q.shape[1]
    a_ref[:, :F] = q
    a_ref[:, F:F + 1] = n
    a_ref[:, F + 1:F + 2] = jnp.ones_like(n)
    c_ref[:, :F] = -2.0 * q
    c_ref[:, F:F + 1] = jnp.ones_like(n)
    c_ref[:, F + 1:F + 2] = n


def _mine_kernel(a_row_ref, c_all_ref, t_row_ref, t_all_ref, out_ref):
    i = pl.program_id(0)
    B = c_all_ref.shape[0]
    R = a_row_ref.shape[0]

    d = jax.lax.dot_general(
        a_row_ref[...], c_all_ref[...],
        dimension_numbers=(((1,), (1,)), ((), ())),
        preferred_element_type=jnp.float32,
    )

    same = t_row_ref[...] == t_all_ref[...]

    inf = jnp.float32(jnp.inf)
    m_pos = jnp.max(jnp.where(same, d, 0.0), axis=1, keepdims=True)
    m_neg = jnp.min(jnp.where(same, inf, d), axis=1, keepdims=True)

    out_ref[...] = jnp.maximum(DELTA - m_pos + m_neg, 0.0)

    F = a_row_ref.shape[1] - 2
    n_row = a_row_ref[:, F:F + 1]
    ok = (jnp.min(m_pos - 1e-3 * n_row) > 0.0) & (jnp.max(m_neg) < inf)

    @pl.when(jnp.logical_not(ok))
    def _exact_fix():
        rid = jax.lax.broadcasted_iota(jnp.int32, (R, 1), 0) + i * R
        col_ids = jax.lax.broadcasted_iota(jnp.int32, (R, B), 1)
        diag = col_ids == rid
        dc = jnp.maximum(d, 0.0)
        big_idx = jnp.int32(B)
        masked_pos = jnp.where(diag, -inf, jnp.where(same, dc, 0.0))
        mp = jnp.max(masked_pos, axis=1, keepdims=True)
        jp = jnp.min(jnp.where(masked_pos == mp, col_ids, big_idx),
                     axis=1, keepdims=True)
        d_pos = jnp.sum(jnp.where(col_ids == jp, dc, 0.0), axis=1,
                        keepdims=True)
        masked_neg = jnp.where(diag | same, inf, dc)
        mn = jnp.min(masked_neg, axis=1, keepdims=True)
        jn = jnp.min(jnp.where(masked_neg == mn, col_ids, big_idx),
                     axis=1, keepdims=True)
        d_neg = jnp.sum(jnp.where(col_ids == jn, dc, 0.0), axis=1,
                        keepdims=True)
        out_ref[...] = jnp.maximum(DELTA - d_pos + d_neg, 0.0)


@jax.jit
def kernel(queries, targets):
    B, F = queries.shape
    FA = F + 2
    t = targets.astype(jnp.int32)
    num_blocks = B // ROW_BLOCK

    a_aug, c_aug = pl.pallas_call(
        _prep_kernel,
        out_shape=(
            jax.ShapeDtypeStruct((B, FA), jnp.float32),
            jax.ShapeDtypeStruct((B, FA), jnp.float32),
        ),
    )(queries)

    per_row = pl.pallas_call(
        _mine_kernel,
        grid=(num_blocks,),
        in_specs=[
            pl.BlockSpec((ROW_BLOCK, FA), lambda i: (i, 0)),
            pl.BlockSpec((B, FA), lambda i: (0, 0)),
            pl.BlockSpec((ROW_BLOCK, 1), lambda i: (i, 0)),
            pl.BlockSpec((1, B), lambda i: (0, 0)),
        ],
        out_specs=pl.BlockSpec((ROW_BLOCK, 1), lambda i: (i, 0)),
        out_shape=jax.ShapeDtypeStruct((B, 1), jnp.float32),
        compiler_params=pltpu.CompilerParams(
            dimension_semantics=("parallel",),
        ),
    )(a_aug, c_aug, t.reshape(B, 1), t.reshape(1, B))

    return jnp.mean(per_row)


# speedup vs baseline: 122.5278x; 1.2172x over previous
import jax
import jax.numpy as jnp
from jax.experimental import pallas as pl
from jax.experimental.pallas import tpu as pltpu

DELTA = 1.0
ROW_BLOCK = 512


def _prep_kernel(q_ref, a_ref, c_ref):
    q = q_ref[...]
    n = jnp.sum(q * q, axis=1, keepdims=True)
    F = q.shape[1]
    a_ref[:, :F] = q
    a_ref[:, F:F + 1] = n
    a_ref[:, F + 1:F + 2] = jnp.ones_like(n)
    c_ref[:, :F] = -2.0 * q
    c_ref[:, F:F + 1] = jnp.ones_like(n)
    c_ref[:, F + 1:F + 2] = n


def _mine_kernel(a_row_ref, c_all_ref, t_row_ref, t_all_ref, out_ref,
                 loss_ref):
    i = pl.program_id(0)
    nb = pl.num_programs(0)
    B = c_all_ref.shape[0]
    R = a_row_ref.shape[0]

    d = jax.lax.dot_general(
        a_row_ref[...], c_all_ref[...],
        dimension_numbers=(((1,), (1,)), ((), ())),
        preferred_element_type=jnp.float32,
    )

    same = t_row_ref[...] == t_all_ref[...]

    inf = jnp.float32(jnp.inf)
    m_pos = jnp.max(jnp.where(same, d, 0.0), axis=1, keepdims=True)
    m_neg = jnp.min(jnp.where(same, inf, d), axis=1, keepdims=True)

    loss_ref[...] = jnp.maximum(DELTA - m_pos + m_neg, 0.0)

    F = a_row_ref.shape[1] - 2
    n_row = a_row_ref[:, F:F + 1]
    ok = (jnp.min(m_pos - 1e-3 * n_row) > 0.0) & (jnp.max(m_neg) < inf)

    @pl.when(jnp.logical_not(ok))
    def _exact_fix():
        rid = jax.lax.broadcasted_iota(jnp.int32, (R, 1), 0) + i * R
        col_ids = jax.lax.broadcasted_iota(jnp.int32, (R, B), 1)
        diag = col_ids == rid
        dc = jnp.maximum(d, 0.0)
        big_idx = jnp.int32(B)
        masked_pos = jnp.where(diag, -inf, jnp.where(same, dc, 0.0))
        mp = jnp.max(masked_pos, axis=1, keepdims=True)
        jp = jnp.min(jnp.where(masked_pos == mp, col_ids, big_idx),
                     axis=1, keepdims=True)
        d_pos = jnp.sum(jnp.where(col_ids == jp, dc, 0.0), axis=1,
                        keepdims=True)
        masked_neg = jnp.where(diag | same, inf, dc)
        mn = jnp.min(masked_neg, axis=1, keepdims=True)
        jn = jnp.min(jnp.where(masked_neg == mn, col_ids, big_idx),
                     axis=1, keepdims=True)
        d_neg = jnp.sum(jnp.where(col_ids == jn, dc, 0.0), axis=1,
                        keepdims=True)
        loss_ref[...] = jnp.maximum(DELTA - d_pos + d_neg, 0.0)

    @pl.when(i == 0)
    def _init():
        out_ref[...] = jnp.zeros((1, 1), jnp.float32)

    out_ref[...] += jnp.sum(loss_ref[...], keepdims=True)

    @pl.when(i == nb - 1)
    def _finish():
        out_ref[...] = out_ref[...] * (1.0 / B)


@jax.jit
def kernel(queries, targets):
    B, F = queries.shape
    FA = F + 2
    t = targets.astype(jnp.int32)
    num_blocks = B // ROW_BLOCK

    a_aug, c_aug = pl.pallas_call(
        _prep_kernel,
        out_shape=(
            jax.ShapeDtypeStruct((B, FA), jnp.float32),
            jax.ShapeDtypeStruct((B, FA), jnp.float32),
        ),
    )(queries)

    loss = pl.pallas_call(
        _mine_kernel,
        grid=(num_blocks,),
        in_specs=[
            pl.BlockSpec((ROW_BLOCK, FA), lambda i: (i, 0)),
            pl.BlockSpec((B, FA), lambda i: (0, 0)),
            pl.BlockSpec((ROW_BLOCK, 1), lambda i: (i, 0)),
            pl.BlockSpec((1, B), lambda i: (0, 0)),
        ],
        out_specs=pl.BlockSpec((1, 1), lambda i: (0, 0)),
        out_shape=jax.ShapeDtypeStruct((1, 1), jnp.float32),
        scratch_shapes=[pltpu.VMEM((ROW_BLOCK, 1), jnp.float32)],
        compiler_params=pltpu.CompilerParams(
            dimension_semantics=("arbitrary",),
        ),
    )(a_aug, c_aug, t.reshape(B, 1), t.reshape(1, B))

    return loss[0, 0]


# speedup vs baseline: 150.2629x; 1.2264x over previous
import jax
import jax.numpy as jnp
from jax.experimental import pallas as pl
from jax.experimental.pallas import tpu as pltpu

DELTA = 1.0
ROW_BLOCK = 512


def _mine_kernel(q_all_ref, t_row_ref, t_all_ref, out_ref,
                 a_ref, c_ref, loss_ref):
    i = pl.program_id(0)
    nb = pl.num_programs(0)
    B, F = q_all_ref.shape
    R = ROW_BLOCK

    @pl.when(i == 0)
    def _build_augmented():
        q = q_all_ref[...]
        n = jnp.sum(q * q, axis=1, keepdims=True)
        a_ref[:, :F] = q
        a_ref[:, F:F + 1] = n
        a_ref[:, F + 1:F + 2] = jnp.ones_like(n)
        c_ref[:, :F] = -2.0 * q
        c_ref[:, F:F + 1] = jnp.ones_like(n)
        c_ref[:, F + 1:F + 2] = n

    a_row = a_ref[pl.ds(i * R, R), :]
    d = jax.lax.dot_general(
        a_row, c_ref[...],
        dimension_numbers=(((1,), (1,)), ((), ())),
        preferred_element_type=jnp.float32,
    )

    same = t_row_ref[...] == t_all_ref[...]

    inf = jnp.float32(jnp.inf)
    m_pos = jnp.max(jnp.where(same, d, 0.0), axis=1, keepdims=True)
    m_neg = jnp.min(jnp.where(same, inf, d), axis=1, keepdims=True)

    loss_ref[...] = jnp.maximum(DELTA - m_pos + m_neg, 0.0)

    n_row = a_ref[pl.ds(i * R, R), F:F + 1]
    ok = (jnp.min(m_pos - 1e-3 * n_row) > 0.0) & (jnp.max(m_neg) < inf)

    @pl.when(jnp.logical_not(ok))
    def _exact_fix():
        rid = jax.lax.broadcasted_iota(jnp.int32, (R, 1), 0) + i * R
        col_ids = jax.lax.broadcasted_iota(jnp.int32, (R, B), 1)
        diag = col_ids == rid
        dc = jnp.maximum(d, 0.0)
        big_idx = jnp.int32(B)
        masked_pos = jnp.where(diag, -inf, jnp.where(same, dc, 0.0))
        mp = jnp.max(masked_pos, axis=1, keepdims=True)
        jp = jnp.min(jnp.where(masked_pos == mp, col_ids, big_idx),
                     axis=1, keepdims=True)
        d_pos = jnp.sum(jnp.where(col_ids == jp, dc, 0.0), axis=1,
                        keepdims=True)
        masked_neg = jnp.where(diag | same, inf, dc)
        mn = jnp.min(masked_neg, axis=1, keepdims=True)
        jn = jnp.min(jnp.where(masked_neg == mn, col_ids, big_idx),
                     axis=1, keepdims=True)
        d_neg = jnp.sum(jnp.where(col_ids == jn, dc, 0.0), axis=1,
                        keepdims=True)
        loss_ref[...] = jnp.maximum(DELTA - d_pos + d_neg, 0.0)

    @pl.when(i == 0)
    def _init():
        out_ref[...] = jnp.zeros((1, 1), jnp.float32)

    out_ref[...] += jnp.sum(loss_ref[...], keepdims=True)

    @pl.when(i == nb - 1)
    def _finish():
        out_ref[...] = out_ref[...] * (1.0 / B)


@jax.jit
def kernel(queries, targets):
    B, F = queries.shape
    t = targets.astype(jnp.int32)
    num_blocks = B // ROW_BLOCK

    loss = pl.pallas_call(
        _mine_kernel,
        grid=(num_blocks,),
        in_specs=[
            pl.BlockSpec((B, F), lambda i: (0, 0)),
            pl.BlockSpec((ROW_BLOCK, 1), lambda i: (i, 0)),
            pl.BlockSpec((1, B), lambda i: (0, 0)),
        ],
        out_specs=pl.BlockSpec((1, 1), lambda i: (0, 0)),
        out_shape=jax.ShapeDtypeStruct((1, 1), jnp.float32),
        scratch_shapes=[
            pltpu.VMEM((B, F + 2), jnp.float32),
            pltpu.VMEM((B, F + 2), jnp.float32),
            pltpu.VMEM((ROW_BLOCK, 1), jnp.float32),
        ],
        compiler_params=pltpu.CompilerParams(
            dimension_semantics=("arbitrary",),
        ),
    )(queries, t.reshape(B, 1), t.reshape(1, B))

    return loss[0, 0]


# speedup vs baseline: 164.3633x; 1.0938x over previous
import jax
import jax.numpy as jnp
from jax.experimental import pallas as pl
from jax.experimental.pallas import tpu as pltpu

DELTA = 1.0
ROW_BLOCK = 512


def _mine_kernel(q_all_ref, t_all_ref, out_ref,
                 a_ref, c_ref, tc_ref, loss_ref):
    i = pl.program_id(0)
    nb = pl.num_programs(0)
    B, F = q_all_ref.shape
    R = ROW_BLOCK

    @pl.when(i == 0)
    def _build_augmented():
        q = q_all_ref[...]
        n = jnp.sum(q * q, axis=1, keepdims=True)
        a_ref[:, :F] = q
        a_ref[:, F:F + 1] = n
        a_ref[:, F + 1:F + 2] = jnp.ones_like(n)
        c_ref[:, :F] = -2.0 * q
        c_ref[:, F:F + 1] = jnp.ones_like(n)
        c_ref[:, F + 1:F + 2] = n
        tc_ref[...] = t_all_ref[...].reshape(B, 1)

    a_row = a_ref[pl.ds(i * R, R), :]
    d = jax.lax.dot_general(
        a_row, c_ref[...],
        dimension_numbers=(((1,), (1,)), ((), ())),
        preferred_element_type=jnp.float32,
    )

    same = tc_ref[pl.ds(i * R, R), :] == t_all_ref[...]

    inf = jnp.float32(jnp.inf)
    m_pos = jnp.max(jnp.where(same, d, 0.0), axis=1, keepdims=True)
    m_neg = jnp.min(jnp.where(same, inf, d), axis=1, keepdims=True)

    loss_ref[...] = jnp.maximum(DELTA - m_pos + m_neg, 0.0)

    n_row = a_ref[pl.ds(i * R, R), F:F + 1]
    ok = (jnp.min(m_pos - 1e-3 * n_row) > 0.0) & (jnp.max(m_neg) < inf)

    @pl.when(jnp.logical_not(ok))
    def _exact_fix():
        rid = jax.lax.broadcasted_iota(jnp.int32, (R, 1), 0) + i * R
        col_ids = jax.lax.broadcasted_iota(jnp.int32, (R, B), 1)
        diag = col_ids == rid
        dc = jnp.maximum(d, 0.0)
        big_idx = jnp.int32(B)
        masked_pos = jnp.where(diag, -inf, jnp.where(same, dc, 0.0))
        mp = jnp.max(masked_pos, axis=1, keepdims=True)
        jp = jnp.min(jnp.where(masked_pos == mp, col_ids, big_idx),
                     axis=1, keepdims=True)
        d_pos = jnp.sum(jnp.where(col_ids == jp, dc, 0.0), axis=1,
                        keepdims=True)
        masked_neg = jnp.where(diag | same, inf, dc)
        mn = jnp.min(masked_neg, axis=1, keepdims=True)
        jn = jnp.min(jnp.where(masked_neg == mn, col_ids, big_idx),
                     axis=1, keepdims=True)
        d_neg = jnp.sum(jnp.where(col_ids == jn, dc, 0.0), axis=1,
                        keepdims=True)
        loss_ref[...] = jnp.maximum(DELTA - d_pos + d_neg, 0.0)

    @pl.when(i == 0)
    def _init():
        out_ref[...] = jnp.zeros((1, 1), jnp.float32)

    out_ref[...] += jnp.sum(loss_ref[...], keepdims=True)

    @pl.when(i == nb - 1)
    def _finish():
        out_ref[...] = out_ref[...] * (1.0 / B)


@jax.jit
def kernel(queries, targets):
    B, F = queries.shape
    t = targets.astype(jnp.int32)
    num_blocks = B // ROW_BLOCK

    loss = pl.pallas_call(
        _mine_kernel,
        grid=(num_blocks,),
        in_specs=[
            pl.BlockSpec((B, F), lambda i: (0, 0)),
            pl.BlockSpec((1, B), lambda i: (0, 0)),
        ],
        out_specs=pl.BlockSpec((1, 1), lambda i: (0, 0)),
        out_shape=jax.ShapeDtypeStruct((1, 1), jnp.float32),
        scratch_shapes=[
            pltpu.VMEM((B, F + 2), jnp.float32),
            pltpu.VMEM((B, F + 2), jnp.float32),
            pltpu.VMEM((B, 1), jnp.int32),
            pltpu.VMEM((ROW_BLOCK, 1), jnp.float32),
        ],
        compiler_params=pltpu.CompilerParams(
            dimension_semantics=("arbitrary",),
        ),
    )(queries, t.reshape(1, B))

    return loss[0, 0]


# speedup vs baseline: 165.0024x; 1.0039x over previous
import jax
import jax.numpy as jnp
from jax.experimental import pallas as pl
from jax.experimental.pallas import tpu as pltpu

DELTA = 1.0
ROW_BLOCK = 512


def _mine_kernel(q_all_ref, t_all_ref, out_ref,
                 a_ref, c_ref, tc_ref, loss_ref):
    i = pl.program_id(0)
    nb = pl.num_programs(0)
    B, F = q_all_ref.shape
    R = ROW_BLOCK

    @pl.when(i == 0)
    def _build_augmented():
        q = q_all_ref[...]
        n = jnp.sum(q * q, axis=1, keepdims=True)
        a_ref[:, :F] = q
        a_ref[:, F:F + 1] = n
        a_ref[:, F + 1:F + 2] = jnp.ones_like(n)
        c_ref[:, :F] = -2.0 * q
        c_ref[:, F:F + 1] = jnp.ones_like(n)
        c_ref[:, F + 1:F + 2] = n
        tc_ref[...] = t_all_ref[...].reshape(B, 1)

    a_row = a_ref[pl.ds(i * R, R), :]
    d = jax.lax.dot_general(
        a_row, c_ref[...],
        dimension_numbers=(((1,), (1,)), ((), ())),
        preferred_element_type=jnp.float32,
    )

    same = tc_ref[pl.ds(i * R, R), :] == t_all_ref[...]

    inf = jnp.float32(jnp.inf)
    m_pos = jnp.max(jnp.where(same, d, 0.0), axis=1, keepdims=True)
    m_neg = jnp.min(jnp.where(same, inf, d), axis=1, keepdims=True)

    loss_ref[...] = jnp.maximum(DELTA - m_pos + m_neg, 0.0)

    n_row = a_ref[pl.ds(i * R, R), F:F + 1]
    ok = (jnp.min(m_pos - 1e-3 * n_row) > 0.0) & (jnp.max(m_neg) < inf)

    @pl.when(jnp.logical_not(ok))
    def _exact_fix():
        rid = jax.lax.broadcasted_iota(jnp.int32, (R, 1), 0) + i * R
        col_ids = jax.lax.broadcasted_iota(jnp.int32, (R, B), 1)
        diag = col_ids == rid
        dc = jnp.maximum(d, 0.0)
        big_idx = jnp.int32(B)
        masked_pos = jnp.where(diag, -inf, jnp.where(same, dc, 0.0))
        mp = jnp.max(masked_pos, axis=1, keepdims=True)
        jp = jnp.min(jnp.where(masked_pos == mp, col_ids, big_idx),
                     axis=1, keepdims=True)
        d_pos = jnp.sum(jnp.where(col_ids == jp, dc, 0.0), axis=1,
                        keepdims=True)
        masked_neg = jnp.where(diag | same, inf, dc)
        mn = jnp.min(masked_neg, axis=1, keepdims=True)
        jn = jnp.min(jnp.where(masked_neg == mn, col_ids, big_idx),
                     axis=1, keepdims=True)
        d_neg = jnp.sum(jnp.where(col_ids == jn, dc, 0.0), axis=1,
                        keepdims=True)
        loss_ref[...] = jnp.maximum(DELTA - d_pos + d_neg, 0.0)

    @pl.when(i == 0)
    def _init():
        out_ref[0] = 0.0

    out_ref[0] += jnp.sum(loss_ref[...])

    @pl.when(i == nb - 1)
    def _finish():
        out_ref[0] = out_ref[0] * (1.0 / B)


@jax.jit
def kernel(queries, targets):
    B, F = queries.shape
    t = targets.astype(jnp.int32)
    num_blocks = B // ROW_BLOCK

    loss = pl.pallas_call(
        _mine_kernel,
        grid=(num_blocks,),
        in_specs=[
            pl.BlockSpec((B, F), lambda i: (0, 0)),
            pl.BlockSpec((1, B), lambda i: (0, 0)),
        ],
        out_specs=pl.BlockSpec(memory_space=pltpu.MemorySpace.SMEM),
        out_shape=jax.ShapeDtypeStruct((1,), jnp.float32),
        scratch_shapes=[
            pltpu.VMEM((B, F + 2), jnp.float32),
            pltpu.VMEM((B, F + 2), jnp.float32),
            pltpu.VMEM((B, 1), jnp.int32),
            pltpu.VMEM((ROW_BLOCK, 1), jnp.float32),
        ],
        compiler_params=pltpu.CompilerParams(
            dimension_semantics=("arbitrary",),
        ),
    )(queries, t.reshape(1, B))

    return loss[0]


# speedup vs baseline: 177.1233x; 1.0735x over previous
import jax
import jax.numpy as jnp
from jax.experimental import pallas as pl
from jax.experimental.pallas import tpu as pltpu

DELTA = 1.0
ROW_BLOCK = 1024


def _mine_kernel(q_all_ref, t_all_ref, out_ref,
                 a_ref, c_ref, tc_ref, loss_ref, d_ref):
    i = pl.program_id(0)
    nb = pl.num_programs(0)
    B, F = q_all_ref.shape
    R = ROW_BLOCK

    @pl.when(i == 0)
    def _build_augmented():
        q = q_all_ref[...]
        n = jnp.sum(q * q, axis=1, keepdims=True)
        a_ref[:, :F] = q
        a_ref[:, F:F + 1] = n
        a_ref[:, F + 1:F + 2] = jnp.ones_like(n)
        c_ref[:, :F] = -2.0 * q
        c_ref[:, F:F + 1] = jnp.ones_like(n)
        c_ref[:, F + 1:F + 2] = n
        tc_ref[...] = t_all_ref[...].reshape(B, 1)

    a_row = a_ref[pl.ds(i * R, R), :]
    d_ref[...] = jax.lax.dot_general(
        a_row, c_ref[...],
        dimension_numbers=(((1,), (1,)), ((), ())),
        preferred_element_type=jnp.float32,
    )
    d = d_ref[...]

    same = tc_ref[pl.ds(i * R, R), :] == t_all_ref[...]

    inf = jnp.float32(jnp.inf)
    m_pos = jnp.max(jnp.where(same, d, 0.0), axis=1, keepdims=True)
    m_neg = jnp.min(jnp.where(same, inf, d), axis=1, keepdims=True)

    loss_ref[...] = jnp.maximum(DELTA - m_pos + m_neg, 0.0)

    n_row = a_ref[pl.ds(i * R, R), F:F + 1]
    ok = (jnp.min(m_pos - 1e-3 * n_row) > 0.0) & (jnp.max(m_neg) < inf)

    @pl.when(jnp.logical_not(ok))
    def _exact_fix():
        rid_l = jax.lax.broadcasted_iota(jnp.int32, (R, 1), 0)
        diag_l = jax.lax.broadcasted_iota(jnp.int32, (R, R), 1) == rid_l
        local = d_ref[:, pl.ds(i * R, R)]
        d_ref[:, pl.ds(i * R, R)] = jnp.where(diag_l, 0.0, local)
        dcz = jnp.maximum(d_ref[...], 0.0)
        mp = jnp.max(jnp.where(same, dcz, 0.0), axis=1, keepdims=True)
        first_row = rid_l + i * R == 0
        d_pos = jnp.where(
            mp > 0.0, mp,
            jnp.where(first_row, dcz[:, 1:2], dcz[:, 0:1]))
        mn = jnp.min(jnp.where(same, inf, dcz), axis=1, keepdims=True)
        d_neg = jnp.where(mn < inf, mn, dcz[:, 0:1])
        loss_ref[...] = jnp.maximum(DELTA - d_pos + d_neg, 0.0)

    @pl.when(i == 0)
    def _init():
        out_ref[0] = 0.0

    out_ref[0] += jnp.sum(loss_ref[...])

    @pl.when(i == nb - 1)
    def _finish():
        out_ref[0] = out_ref[0] * (1.0 / B)


@jax.jit
def kernel(queries, targets):
    B, F = queries.shape
    t = targets.astype(jnp.int32)
    num_blocks = B // ROW_BLOCK

    loss = pl.pallas_call(
        _mine_kernel,
        grid=(num_blocks,),
        in_specs=[
            pl.BlockSpec((B, F), lambda i: (0, 0)),
            pl.BlockSpec((1, B), lambda i: (0, 0)),
        ],
        out_specs=pl.BlockSpec(memory_space=pltpu.MemorySpace.SMEM),
        out_shape=jax.ShapeDtypeStruct((1,), jnp.float32),
        scratch_shapes=[
            pltpu.VMEM((B, F + 2), jnp.float32),
            pltpu.VMEM((B, F + 2), jnp.float32),
            pltpu.VMEM((B, 1), jnp.int32),
            pltpu.VMEM((ROW_BLOCK, 1), jnp.float32),
            pltpu.VMEM((ROW_BLOCK, B), jnp.float32),
        ],
        compiler_params=pltpu.CompilerParams(
            dimension_semantics=("arbitrary",),
        ),
    )(queries, t.reshape(1, B))

    return loss[0]
